# Initial kernel scaffold; baseline (speedup 1.0000x reference)
import jax
import jax.numpy as jnp
from jax.experimental import pallas as pl


def kernel(ts_out, seq_out, omega, patch_mask):
    raise NotImplementedError("write your pallas kernel here")


# speedup vs baseline: 2.6252x; 2.6252x over previous
import jax
import jax.numpy as jnp
from jax.experimental import pallas as pl
from jax.experimental.pallas import tpu as pltpu

_TAU = 0.02
_INV_TAU = 1.0 / _TAU
_EPS = 1e-12

_N = 8192
_D = 128
_B = 256
_BP = 512


def _prep_kernel(sq_ref, out_ref):
    x = sq_ref[...]
    ssq = jnp.sum(x * x, axis=1, keepdims=True)
    inv = 1.0 / jnp.maximum(jnp.sqrt(ssq), _EPS)
    out_ref[...] = (x * inv).astype(jnp.bfloat16)


def _loss_kernel(ts_ref, sqn_ref, pm_ref, out_ref):
    i = pl.program_id(0)
    t = ts_ref[...]
    ssq = jnp.sum(t * t, axis=1, keepdims=True)
    inv = _INV_TAU / jnp.maximum(jnp.sqrt(ssq), _EPS)
    tb = (t * inv).astype(jnp.bfloat16)

    x = jax.lax.dot_general(
        tb, sqn_ref[...],
        dimension_numbers=(((1,), (1,)), ((), ())),
        preferred_element_type=jnp.float32,
    )

    m = jnp.max(x, axis=1, keepdims=True)
    s = jnp.sum(jnp.exp(x - m), axis=1, keepdims=True)
    lse = m + jnp.log(s)

    sqd = sqn_ref[pl.ds(i * _B, _B), :].astype(jnp.float32)
    diag = jnp.sum(tb.astype(jnp.float32) * sqd, axis=1, keepdims=True)

    pm = pm_ref[0]
    lp = jax.lax.dot_general(
        pm, diag - lse,
        dimension_numbers=(((1,), (0,)), ((), ())),
        preferred_element_type=jnp.float32,
        precision=jax.lax.Precision.HIGHEST,
    )
    out_ref[0, 0, 0] = lp[0, 0]
    out_ref[0, 0, 1] = jnp.sum(pm)


def kernel(ts_out, seq_out, omega, patch_mask):
    del omega
    n, d = ts_out.shape

    sq_n = pl.pallas_call(
        _prep_kernel,
        grid=(n // _BP,),
        in_specs=[pl.BlockSpec((_BP, d), lambda i: (i, 0))],
        out_specs=pl.BlockSpec((_BP, d), lambda i: (i, 0)),
        out_shape=jax.ShapeDtypeStruct((n, d), jnp.bfloat16),
        compiler_params=pltpu.CompilerParams(
            dimension_semantics=("parallel",),
        ),
        name="nce_normalize",
    )(seq_out)

    g = n // _B
    pmf = patch_mask.astype(jnp.float32).reshape(g, 1, _B)

    parts = pl.pallas_call(
        _loss_kernel,
        grid=(g,),
        in_specs=[
            pl.BlockSpec((_B, d), lambda i: (i, 0)),
            pl.BlockSpec((n, d), lambda i: (0, 0)),
            pl.BlockSpec((1, 1, _B), lambda i: (i, 0, 0)),
        ],
        out_specs=pl.BlockSpec((1, 1, 2), lambda i: (i, 0, 0), memory_space=pltpu.SMEM),
        out_shape=jax.ShapeDtypeStruct((g, 1, 2), jnp.float32),
        compiler_params=pltpu.CompilerParams(
            dimension_semantics=("parallel",),
            vmem_limit_bytes=100 * 1024 * 1024,
        ),
        name="nce_loss",
    )(ts_out, sq_n, pmf)

    return -jnp.sum(parts[:, 0, 0]) / (jnp.sum(parts[:, 0, 1]) + 1e-6)


# speedup vs baseline: 3.9504x; 1.5048x over previous
import jax
import jax.numpy as jnp
from jax.experimental import pallas as pl
from jax.experimental.pallas import tpu as pltpu

_TAU = 0.02
_INV_TAU = 1.0 / _TAU
_EPS = 1e-12

_N = 8192
_D = 128
_B = 256
_BP = 512


def _prep_kernel(sq_ref, out_ref):
    x = sq_ref[...]
    ssq = jnp.sum(x * x, axis=1, keepdims=True)
    inv = 1.0 / jnp.maximum(jnp.sqrt(ssq), _EPS)
    out_ref[...] = (x * inv).astype(jnp.bfloat16)


def _loss_kernel(ts_ref, sqn_ref, pm_ref, out_ref):
    i = pl.program_id(0)
    t = ts_ref[...]
    ssq = jnp.sum(t * t, axis=1, keepdims=True)
    inv = _INV_TAU / jnp.maximum(jnp.sqrt(ssq), _EPS)
    tb = (t * inv).astype(jnp.bfloat16)

    x = jax.lax.dot_general(
        tb, sqn_ref[...],
        dimension_numbers=(((1,), (1,)), ((), ())),
        preferred_element_type=jnp.float32,
    )

    s = jnp.sum(jnp.exp(x), axis=1, keepdims=True)
    lse = jnp.log(s)

    sqd = sqn_ref[pl.ds(i * _B, _B), :].astype(jnp.float32)
    diag = jnp.sum(tb.astype(jnp.float32) * sqd, axis=1, keepdims=True)

    pm = pm_ref[0]
    lp = jax.lax.dot_general(
        pm, diag - lse,
        dimension_numbers=(((1,), (0,)), ((), ())),
        preferred_element_type=jnp.float32,
        precision=jax.lax.Precision.HIGHEST,
    )
    out_ref[0, 0, 0] = lp[0, 0]
    out_ref[0, 0, 1] = jnp.sum(pm)


def kernel(ts_out, seq_out, omega, patch_mask):
    del omega
    n, d = ts_out.shape

    sq_n = pl.pallas_call(
        _prep_kernel,
        grid=(n // _BP,),
        in_specs=[pl.BlockSpec((_BP, d), lambda i: (i, 0))],
        out_specs=pl.BlockSpec((_BP, d), lambda i: (i, 0)),
        out_shape=jax.ShapeDtypeStruct((n, d), jnp.bfloat16),
        compiler_params=pltpu.CompilerParams(
            dimension_semantics=("parallel",),
        ),
        name="nce_normalize",
    )(seq_out)

    g = n // _B
    pmf = patch_mask.astype(jnp.float32).reshape(g, 1, _B)

    parts = pl.pallas_call(
        _loss_kernel,
        grid=(g,),
        in_specs=[
            pl.BlockSpec((_B, d), lambda i: (i, 0)),
            pl.BlockSpec((n, d), lambda i: (0, 0)),
            pl.BlockSpec((1, 1, _B), lambda i: (i, 0, 0)),
        ],
        out_specs=pl.BlockSpec((1, 1, 2), lambda i: (i, 0, 0),
                               memory_space=pltpu.SMEM),
        out_shape=jax.ShapeDtypeStruct((g, 1, 2), jnp.float32),
        compiler_params=pltpu.CompilerParams(
            dimension_semantics=("parallel",),
            vmem_limit_bytes=100 * 1024 * 1024,
        ),
        name="nce_loss",
    )(ts_out, sq_n, pmf)

    return -jnp.sum(parts[:, 0, 0]) / (jnp.sum(parts[:, 0, 1]) + 1e-6)


# speedup vs baseline: 4.7854x; 1.2114x over previous
import jax
import jax.numpy as jnp
from jax.experimental import pallas as pl
from jax.experimental.pallas import tpu as pltpu

_TAU = 0.02
_LOG2E = 1.4426950408889634
_LN2 = 0.6931471805599453
_SCALE = _LOG2E / _TAU
_EPS = 1e-12

_N = 8192
_D = 128
_B = 512
_BP = 1024


def _prep_kernel(sq_ref, out_ref):
    x = sq_ref[...]
    ssq = jnp.sum(x * x, axis=1, keepdims=True)
    inv = 1.0 / jnp.maximum(jnp.sqrt(ssq), _EPS)
    out_ref[...] = (x * inv).astype(jnp.bfloat16)


def _loss_kernel(ts_ref, sqn_ref, pm_ref, out_ref):
    i = pl.program_id(0)
    t = ts_ref[...]
    ssq = jnp.sum(t * t, axis=1, keepdims=True)
    inv = _SCALE / jnp.maximum(jnp.sqrt(ssq), _EPS)
    tb = (t * inv).astype(jnp.bfloat16)

    x = jax.lax.dot_general(
        tb, sqn_ref[...],
        dimension_numbers=(((1,), (1,)), ((), ())),
        preferred_element_type=jnp.float32,
    )

    s = jnp.sum(jnp.exp2(x), axis=1, keepdims=True)
    lse = jnp.log(s)

    sqd = sqn_ref[pl.ds(i * _B, _B), :].astype(jnp.float32)
    diag = jnp.sum(tb.astype(jnp.float32) * sqd, axis=1, keepdims=True) * _LN2

    pm = pm_ref[0]
    lp = jax.lax.dot_general(
        pm, diag - lse,
        dimension_numbers=(((1,), (0,)), ((), ())),
        preferred_element_type=jnp.float32,
        precision=jax.lax.Precision.HIGHEST,
    )
    out_ref[0, 0, 0] = lp[0, 0]
    out_ref[0, 0, 1] = jnp.sum(pm)


def kernel(ts_out, seq_out, omega, patch_mask):
    del omega
    n, d = ts_out.shape

    sq_n = pl.pallas_call(
        _prep_kernel,
        grid=(n // _BP,),
        in_specs=[pl.BlockSpec((_BP, d), lambda i: (i, 0))],
        out_specs=pl.BlockSpec((_BP, d), lambda i: (i, 0)),
        out_shape=jax.ShapeDtypeStruct((n, d), jnp.bfloat16),
        compiler_params=pltpu.CompilerParams(
            dimension_semantics=("parallel",),
        ),
        name="nce_normalize",
    )(seq_out)

    g = n // _B
    pmf = patch_mask.astype(jnp.float32).reshape(g, 1, _B)

    parts = pl.pallas_call(
        _loss_kernel,
        grid=(g,),
        in_specs=[
            pl.BlockSpec((_B, d), lambda i: (i, 0)),
            pl.BlockSpec((n, d), lambda i: (0, 0)),
            pl.BlockSpec((1, 1, _B), lambda i: (i, 0, 0)),
        ],
        out_specs=pl.BlockSpec((1, 1, 2), lambda i: (i, 0, 0),
                               memory_space=pltpu.SMEM),
        out_shape=jax.ShapeDtypeStruct((g, 1, 2), jnp.float32),
        compiler_params=pltpu.CompilerParams(
            dimension_semantics=("parallel",),
            vmem_limit_bytes=100 * 1024 * 1024,
        ),
        name="nce_loss",
    )(ts_out, sq_n, pmf)

    return -jnp.sum(parts[:, 0, 0]) / (jnp.sum(parts[:, 0, 1]) + 1e-6)


# speedup vs baseline: 5.4966x; 1.1486x over previous
import jax
import jax.numpy as jnp
from jax.experimental import pallas as pl
from jax.experimental.pallas import tpu as pltpu

_TAU = 0.02
_LOG2E = 1.4426950408889634
_LN2 = 0.6931471805599453
_SCALE = _LOG2E / _TAU
_EPS = 1e-12

_B = 1024
_BP = 1024


def _prep_kernel(sq_ref, out_ref):
    x = sq_ref[...]
    ssq = jnp.sum(x * x, axis=1, keepdims=True)
    inv = 1.0 / jnp.maximum(jnp.sqrt(ssq), _EPS)
    out_ref[...] = (x * inv).astype(jnp.bfloat16)


def _loss_kernel(ts_ref, sqn_ref, pm_ref, out_ref, acc_ref):
    i = pl.program_id(0)
    ng = pl.num_programs(0)
    t = ts_ref[...]
    ssq = jnp.sum(t * t, axis=1, keepdims=True)
    inv = _SCALE / jnp.maximum(jnp.sqrt(ssq), _EPS)
    tb = (t * inv).astype(jnp.bfloat16)

    x = jax.lax.dot_general(
        tb, sqn_ref[...],
        dimension_numbers=(((1,), (1,)), ((), ())),
        preferred_element_type=jnp.float32,
    )

    s = jnp.sum(jnp.exp2(x), axis=1, keepdims=True)
    lse = jnp.log(s)

    sqd = sqn_ref[pl.ds(i * _B, _B), :].astype(jnp.float32)
    diag = jnp.sum(tb.astype(jnp.float32) * sqd, axis=1, keepdims=True) * _LN2

    pm = pm_ref[0].astype(jnp.float32)
    lp = jax.lax.dot_general(
        pm, diag - lse,
        dimension_numbers=(((1,), (0,)), ((), ())),
        preferred_element_type=jnp.float32,
        precision=jax.lax.Precision.HIGHEST,
    )

    @pl.when(i == 0)
    def _():
        acc_ref[0] = 0.0
        acc_ref[1] = 0.0

    acc_ref[0] += lp[0, 0]
    acc_ref[1] += jnp.sum(pm)

    @pl.when(i == ng - 1)
    def _():
        out_ref[0, 0] = -acc_ref[0] / (acc_ref[1] + 1e-6)


def kernel(ts_out, seq_out, omega, patch_mask):
    del omega
    n, d = ts_out.shape

    sq_n = pl.pallas_call(
        _prep_kernel,
        grid=(n // _BP,),
        in_specs=[pl.BlockSpec((_BP, d), lambda i: (i, 0))],
        out_specs=pl.BlockSpec((_BP, d), lambda i: (i, 0)),
        out_shape=jax.ShapeDtypeStruct((n, d), jnp.bfloat16),
        compiler_params=pltpu.CompilerParams(
            dimension_semantics=("parallel",),
        ),
        name="nce_normalize",
    )(seq_out)

    g = n // _B
    pm3 = patch_mask.reshape(g, 1, _B)

    loss = pl.pallas_call(
        _loss_kernel,
        grid=(g,),
        in_specs=[
            pl.BlockSpec((_B, d), lambda i: (i, 0)),
            pl.BlockSpec((n, d), lambda i: (0, 0)),
            pl.BlockSpec((1, 1, _B), lambda i: (i, 0, 0)),
        ],
        out_specs=pl.BlockSpec((1, 1), lambda i: (0, 0),
                               memory_space=pltpu.SMEM),
        out_shape=jax.ShapeDtypeStruct((1, 1), jnp.float32),
        scratch_shapes=[pltpu.SMEM((2,), jnp.float32)],
        compiler_params=pltpu.CompilerParams(
            dimension_semantics=("arbitrary",),
            vmem_limit_bytes=100 * 1024 * 1024,
        ),
        name="nce_loss",
    )(ts_out, sq_n, pm3)

    return loss[0, 0]


# speedup vs baseline: 6.1367x; 1.1165x over previous
import jax
import jax.numpy as jnp
from jax.experimental import pallas as pl
from jax.experimental.pallas import tpu as pltpu

_TAU = 0.02
_LOG2E = 1.4426950408889634
_LN2 = 0.6931471805599453
_SCALE = _LOG2E / _TAU
_EPS = 1e-12

_B = 1024


def _loss_kernel(ts_ref, sq_ref, pm_ref, out_ref, sqn_ref, acc_ref):
    i = pl.program_id(0)
    ng = pl.num_programs(0)

    @pl.when(i == 0)
    def _():
        q = sq_ref[...]
        qs = jnp.sum(q * q, axis=1, keepdims=True)
        qinv = 1.0 / jnp.maximum(jnp.sqrt(qs), _EPS)
        sqn_ref[...] = (q * qinv).astype(jnp.bfloat16)
        acc_ref[0] = 0.0
        acc_ref[1] = 0.0

    t = ts_ref[...]
    ssq = jnp.sum(t * t, axis=1, keepdims=True)
    inv = _SCALE / jnp.maximum(jnp.sqrt(ssq), _EPS)
    tb = (t * inv).astype(jnp.bfloat16)

    x = jax.lax.dot_general(
        tb, sqn_ref[...],
        dimension_numbers=(((1,), (1,)), ((), ())),
        preferred_element_type=jnp.float32,
    )

    s = jnp.sum(jnp.exp2(x), axis=1, keepdims=True)
    lse = jnp.log(s)

    sqd = sqn_ref[pl.ds(i * _B, _B), :].astype(jnp.float32)
    diag = jnp.sum(tb.astype(jnp.float32) * sqd, axis=1, keepdims=True) * _LN2

    pm = pm_ref[0].astype(jnp.float32)
    lp = jax.lax.dot_general(
        pm, diag - lse,
        dimension_numbers=(((1,), (0,)), ((), ())),
        preferred_element_type=jnp.float32,
        precision=jax.lax.Precision.HIGHEST,
    )

    acc_ref[0] += lp[0, 0]
    acc_ref[1] += jnp.sum(pm)

    @pl.when(i == ng - 1)
    def _():
        out_ref[0, 0] = -acc_ref[0] / (acc_ref[1] + 1e-6)


def kernel(ts_out, seq_out, omega, patch_mask):
    del omega
    n, d = ts_out.shape
    g = n // _B
    pm3 = patch_mask.reshape(g, 1, _B)

    loss = pl.pallas_call(
        _loss_kernel,
        grid=(g,),
        in_specs=[
            pl.BlockSpec((_B, d), lambda i: (i, 0)),
            pl.BlockSpec((n, d), lambda i: (0, 0)),
            pl.BlockSpec((1, 1, _B), lambda i: (i, 0, 0)),
        ],
        out_specs=pl.BlockSpec((1, 1), lambda i: (0, 0),
                               memory_space=pltpu.SMEM),
        out_shape=jax.ShapeDtypeStruct((1, 1), jnp.float32),
        scratch_shapes=[
            pltpu.VMEM((n, d), jnp.bfloat16),
            pltpu.SMEM((2,), jnp.float32),
        ],
        compiler_params=pltpu.CompilerParams(
            dimension_semantics=("arbitrary",),
            vmem_limit_bytes=100 * 1024 * 1024,
        ),
        name="nce_loss",
    )(ts_out, seq_out, pm3)

    return loss[0, 0]


# speedup vs baseline: 6.2016x; 1.0106x over previous
import jax
import jax.numpy as jnp
from jax.experimental import pallas as pl
from jax.experimental.pallas import tpu as pltpu

_TAU = 0.02
_LOG2E = 1.4426950408889634
_LN2 = 0.6931471805599453
_SCALE = _LOG2E / _TAU
_EPS = 1e-12

_B = 2048


def _loss_kernel(ts_ref, sq_ref, pm_ref, out_ref, sqn_ref, acc_ref):
    i = pl.program_id(0)
    ng = pl.num_programs(0)

    @pl.when(i == 0)
    def _():
        q = sq_ref[...]
        qs = jnp.sum(q * q, axis=1, keepdims=True)
        qinv = 1.0 / jnp.maximum(jnp.sqrt(qs), _EPS)
        sqn_ref[...] = (q * qinv).astype(jnp.bfloat16)
        acc_ref[0] = 0.0
        acc_ref[1] = 0.0

    t = ts_ref[...]
    ssq = jnp.sum(t * t, axis=1, keepdims=True)
    inv = _SCALE / jnp.maximum(jnp.sqrt(ssq), _EPS)
    tb = (t * inv).astype(jnp.bfloat16)

    x = jax.lax.dot_general(
        tb, sqn_ref[...],
        dimension_numbers=(((1,), (1,)), ((), ())),
        preferred_element_type=jnp.float32,
    )

    s = jnp.sum(jnp.exp2(x), axis=1, keepdims=True)
    lse = jnp.log(s)

    sqd = sqn_ref[pl.ds(i * _B, _B), :].astype(jnp.float32)
    diag = jnp.sum(tb.astype(jnp.float32) * sqd, axis=1, keepdims=True) * _LN2

    pm = pm_ref[0].astype(jnp.float32)
    lp = jax.lax.dot_general(
        pm, diag - lse,
        dimension_numbers=(((1,), (0,)), ((), ())),
        preferred_element_type=jnp.float32,
        precision=jax.lax.Precision.HIGHEST,
    )

    acc_ref[0] += lp[0, 0]
    acc_ref[1] += jnp.sum(pm)

    @pl.when(i == ng - 1)
    def _():
        out_ref[0, 0] = -acc_ref[0] / (acc_ref[1] + 1e-6)


def kernel(ts_out, seq_out, omega, patch_mask):
    del omega
    n, d = ts_out.shape
    g = n // _B
    pm3 = patch_mask.reshape(g, 1, _B)

    loss = pl.pallas_call(
        _loss_kernel,
        grid=(g,),
        in_specs=[
            pl.BlockSpec((_B, d), lambda i: (i, 0)),
            pl.BlockSpec((n, d), lambda i: (0, 0)),
            pl.BlockSpec((1, 1, _B), lambda i: (i, 0, 0)),
        ],
        out_specs=pl.BlockSpec((1, 1), lambda i: (0, 0),
                               memory_space=pltpu.SMEM),
        out_shape=jax.ShapeDtypeStruct((1, 1), jnp.float32),
        scratch_shapes=[
            pltpu.VMEM((n, d), jnp.bfloat16),
            pltpu.SMEM((2,), jnp.float32),
        ],
        compiler_params=pltpu.CompilerParams(
            dimension_semantics=("arbitrary",),
            vmem_limit_bytes=120 * 1024 * 1024,
        ),
        name="nce_loss",
    )(ts_out, seq_out, pm3)

    return loss[0, 0]
